# Initial kernel scaffold; baseline (speedup 1.0000x reference)
import jax
import jax.numpy as jnp
from jax.experimental import pallas as pl


def kernel(pointcloud, params):
    raise NotImplementedError("write your pallas kernel here")


# speedup vs baseline: 1.0643x; 1.0643x over previous
import functools

import jax
import jax.numpy as jnp
from jax import lax
from jax.experimental import pallas as pl
from jax.experimental.pallas import tpu as pltpu


def _fps_body(x_ref, out_ref, acc_ref, *, n, s):
    x = x_ref[0]
    x0 = x[0:1, :]
    x1 = x[1:2, :]
    x2 = x[2:3, :]
    iota_n = lax.broadcasted_iota(jnp.int32, (1, n), 1)
    iota_s = lax.broadcasted_iota(jnp.int32, (1, s), 1)
    acc_ref[...] = jnp.zeros((3, s), jnp.float32)

    def step(t, carry):
        dist, far = carry
        onehot = iota_n == far
        c0 = jnp.sum(jnp.where(onehot, x0, 0.0), axis=1, keepdims=True)
        c1 = jnp.sum(jnp.where(onehot, x1, 0.0), axis=1, keepdims=True)
        c2 = jnp.sum(jnp.where(onehot, x2, 0.0), axis=1, keepdims=True)
        sel = (iota_s == t).astype(jnp.float32)
        acc_ref[0:1, :] += c0 * sel
        acc_ref[1:2, :] += c1 * sel
        acc_ref[2:3, :] += c2 * sel
        d = (x0 - c0) ** 2 + (x1 - c1) ** 2 + (x2 - c2) ** 2
        dist = jnp.minimum(dist, d)
        far = jnp.argmax(dist, axis=1).astype(jnp.int32).reshape(1, 1)
        return dist, far

    init = (jnp.full((1, n), 1e10, jnp.float32), jnp.zeros((1, 1), jnp.int32))
    lax.fori_loop(0, s, step, init)
    out_ref[0] = acc_ref[...]


def _fps_pallas(xyz, s):
    b, n, _ = xyz.shape
    xyzt = jnp.transpose(xyz, (0, 2, 1))
    out = pl.pallas_call(
        functools.partial(_fps_body, n=n, s=s),
        grid=(b,),
        in_specs=[pl.BlockSpec((1, 3, n), lambda i: (i, 0, 0))],
        out_specs=pl.BlockSpec((1, 3, s), lambda i: (i, 0, 0)),
        out_shape=jax.ShapeDtypeStruct((b, 3, s), jnp.float32),
        scratch_shapes=[pltpu.VMEM((3, s), jnp.float32)],
    )(xyzt)
    return jnp.transpose(out, (0, 2, 1))


def _index_points(points, idx):
    def g(p, i):
        return p[i]

    return jax.vmap(g)(points, idx)


def _ball_query(radius, nsample, xyz, new_xyz):
    b, n, _ = xyz.shape
    s = new_xyz.shape[1]
    sqr = jnp.sum((new_xyz[:, :, None, :] - xyz[:, None, :, :]) ** 2, axis=-1)
    idx = jnp.broadcast_to(jnp.arange(n, dtype=jnp.int32), (b, s, n))
    idx = jnp.where(sqr > radius * radius, n, idx)
    idx = jnp.sort(idx, axis=-1)[:, :, :nsample]
    first = idx[:, :, :1]
    idx = jnp.where(idx == n, jnp.broadcast_to(first, idx.shape), idx)
    return idx


def _mlp(x, layers):
    for (w, g, bta) in layers:
        x = jnp.einsum('...c,cd->...d', x, w)
        m = jnp.mean(x, axis=tuple(range(x.ndim - 1)), keepdims=True)
        v = jnp.var(x, axis=tuple(range(x.ndim - 1)), keepdims=True)
        x = (x - m) / jnp.sqrt(v + 1e-5) * g + bta
        x = jax.nn.relu(x)
    return x


def _sa(xyz, feats, npoint, radius, nsample, layers):
    if npoint is None:
        grouped = xyz[:, None, :, :]
        if feats is not None:
            grouped = jnp.concatenate([grouped, feats[:, None, :, :]], axis=-1)
        new_xyz = jnp.zeros((xyz.shape[0], 1, 3), dtype=xyz.dtype)
    else:
        new_xyz = _fps_pallas(xyz, npoint)
        gidx = _ball_query(radius, nsample, xyz, new_xyz)
        grouped = _index_points(xyz, gidx) - new_xyz[:, :, None, :]
        if feats is not None:
            grouped = jnp.concatenate([grouped, _index_points(feats, gidx)], axis=-1)
    out = _mlp(grouped, layers)
    return new_xyz, jnp.max(out, axis=2)


def kernel(pointcloud, params):
    xyz = pointcloud[..., :3]
    feats = pointcloud[..., 3:]
    xyz, feats = _sa(xyz, feats, 512, 0.2, 64, params['sa1'])
    xyz, feats = _sa(xyz, feats, 128, 0.4, 64, params['sa2'])
    xyz, feats = _sa(xyz, feats, None, None, None, params['sa3'])
    x = feats[:, 0, :]
    w1, g1, b1, w2, g2, b2, w3, b3 = params['fc']
    x = x @ w1
    m = jnp.mean(x, axis=0, keepdims=True)
    v = jnp.var(x, axis=0, keepdims=True)
    x = jax.nn.relu((x - m) / jnp.sqrt(v + 1e-5) * g1 + b1)
    x = x @ w2
    m = jnp.mean(x, axis=0, keepdims=True)
    v = jnp.var(x, axis=0, keepdims=True)
    x = jax.nn.relu((x - m) / jnp.sqrt(v + 1e-5) * g2 + b2)
    return x @ w3 + b3


# speedup vs baseline: 1.0837x; 1.0182x over previous
import functools

import jax
import jax.numpy as jnp
from jax import lax
from jax.experimental import pallas as pl
from jax.experimental.pallas import tpu as pltpu


def _fps_body(x_ref, out_ref, acc_ref, *, n, s):
    x = x_ref[0]
    x0 = x[0:1, :]
    x1 = x[1:2, :]
    x2 = x[2:3, :]
    iota_n = lax.broadcasted_iota(jnp.int32, (1, n), 1)
    iota_s = lax.broadcasted_iota(jnp.int32, (1, s), 1)
    acc_ref[...] = jnp.zeros((3, s), jnp.float32)

    def step(t, carry):
        dist, far = carry
        onehot = iota_n == far
        c0 = jnp.sum(jnp.where(onehot, x0, 0.0), axis=1, keepdims=True)
        c1 = jnp.sum(jnp.where(onehot, x1, 0.0), axis=1, keepdims=True)
        c2 = jnp.sum(jnp.where(onehot, x2, 0.0), axis=1, keepdims=True)
        sel = (iota_s == t).astype(jnp.float32)
        acc_ref[0:1, :] += c0 * sel
        acc_ref[1:2, :] += c1 * sel
        acc_ref[2:3, :] += c2 * sel
        d = (x0 - c0) ** 2 + (x1 - c1) ** 2 + (x2 - c2) ** 2
        dist = jnp.minimum(dist, d)
        far = jnp.argmax(dist, axis=1).astype(jnp.int32).reshape(1, 1)
        return dist, far

    init = (jnp.full((1, n), 1e10, jnp.float32), jnp.zeros((1, 1), jnp.int32))
    lax.fori_loop(0, s, step, init)
    out_ref[0] = acc_ref[...]


def _fps_pallas(xyz, s):
    b, n, _ = xyz.shape
    xyzt = jnp.transpose(xyz, (0, 2, 1))
    out = pl.pallas_call(
        functools.partial(_fps_body, n=n, s=s),
        grid=(b,),
        in_specs=[pl.BlockSpec((1, 3, n), lambda i: (i, 0, 0))],
        out_specs=pl.BlockSpec((1, 3, s), lambda i: (i, 0, 0)),
        out_shape=jax.ShapeDtypeStruct((b, 3, s), jnp.float32),
        scratch_shapes=[pltpu.VMEM((3, s), jnp.float32)],
    )(xyzt)
    return jnp.transpose(out, (0, 2, 1))


def _bq_body(nx_ref, xt_ref, out_ref, *, n, bs, nsample, r2):
    c = nx_ref[0]
    x = xt_ref[0]
    d = ((c[:, 0:1] - x[0:1, :]) ** 2
         + (c[:, 1:2] - x[1:2, :]) ** 2
         + (c[:, 2:3] - x[2:3, :]) ** 2)
    mask = d <= r2
    mi = mask.astype(jnp.int32)

    rank = mi
    sh = 1
    while sh < n:
        shifted = jnp.concatenate(
            [jnp.zeros((bs, sh), jnp.int32), rank[:, :n - sh]], axis=1)
        rank = rank + shifted
        sh *= 2

    cnt = rank[:, n - 1:n]
    iota_s = lax.broadcasted_iota(jnp.int32, (1, nsample), 1)
    tr = jnp.where(iota_s < cnt, iota_s + 1, 1)

    sel = (rank[:, None, :] == tr[:, :, None]) & mask[:, None, :]
    iota_n = lax.broadcasted_iota(jnp.int32, (bs, nsample, n), 2)
    out_ref[0] = jnp.sum(jnp.where(sel, iota_n, 0), axis=2)


def _ball_query_pallas(radius, nsample, xyz, new_xyz):
    b, n, _ = xyz.shape
    s = new_xyz.shape[1]
    bs = 8
    import numpy as _np
    r2 = float(_np.float32(radius) * _np.float32(radius))
    xyzt = jnp.transpose(xyz, (0, 2, 1))
    return pl.pallas_call(
        functools.partial(_bq_body, n=n, bs=bs, nsample=nsample, r2=r2),
        grid=(b, s // bs),
        in_specs=[
            pl.BlockSpec((1, bs, 3), lambda i, j: (i, j, 0)),
            pl.BlockSpec((1, 3, n), lambda i, j: (i, 0, 0)),
        ],
        out_specs=pl.BlockSpec((1, bs, nsample), lambda i, j: (i, j, 0)),
        out_shape=jax.ShapeDtypeStruct((b, s, nsample), jnp.int32),
    )(new_xyz, xyzt)


def _index_points(points, idx):
    def g(p, i):
        return p[i]

    return jax.vmap(g)(points, idx)


def _ball_query(radius, nsample, xyz, new_xyz):
    b, n, _ = xyz.shape
    s = new_xyz.shape[1]
    sqr = jnp.sum((new_xyz[:, :, None, :] - xyz[:, None, :, :]) ** 2, axis=-1)
    idx = jnp.broadcast_to(jnp.arange(n, dtype=jnp.int32), (b, s, n))
    idx = jnp.where(sqr > radius * radius, n, idx)
    idx = jnp.sort(idx, axis=-1)[:, :, :nsample]
    first = idx[:, :, :1]
    idx = jnp.where(idx == n, jnp.broadcast_to(first, idx.shape), idx)
    return idx


def _mlp(x, layers):
    for (w, g, bta) in layers:
        x = jnp.einsum('...c,cd->...d', x, w)
        m = jnp.mean(x, axis=tuple(range(x.ndim - 1)), keepdims=True)
        v = jnp.var(x, axis=tuple(range(x.ndim - 1)), keepdims=True)
        x = (x - m) / jnp.sqrt(v + 1e-5) * g + bta
        x = jax.nn.relu(x)
    return x


def _sa(xyz, feats, npoint, radius, nsample, layers):
    if npoint is None:
        grouped = xyz[:, None, :, :]
        if feats is not None:
            grouped = jnp.concatenate([grouped, feats[:, None, :, :]], axis=-1)
        new_xyz = jnp.zeros((xyz.shape[0], 1, 3), dtype=xyz.dtype)
    else:
        new_xyz = _fps_pallas(xyz, npoint)
        gidx = _ball_query_pallas(radius, nsample, xyz, new_xyz)
        grouped = _index_points(xyz, gidx) - new_xyz[:, :, None, :]
        if feats is not None:
            grouped = jnp.concatenate([grouped, _index_points(feats, gidx)], axis=-1)
    out = _mlp(grouped, layers)
    return new_xyz, jnp.max(out, axis=2)


def kernel(pointcloud, params):
    xyz = pointcloud[..., :3]
    feats = pointcloud[..., 3:]
    xyz, feats = _sa(xyz, feats, 512, 0.2, 64, params['sa1'])
    xyz, feats = _sa(xyz, feats, 128, 0.4, 64, params['sa2'])
    xyz, feats = _sa(xyz, feats, None, None, None, params['sa3'])
    x = feats[:, 0, :]
    w1, g1, b1, w2, g2, b2, w3, b3 = params['fc']
    x = x @ w1
    m = jnp.mean(x, axis=0, keepdims=True)
    v = jnp.var(x, axis=0, keepdims=True)
    x = jax.nn.relu((x - m) / jnp.sqrt(v + 1e-5) * g1 + b1)
    x = x @ w2
    m = jnp.mean(x, axis=0, keepdims=True)
    v = jnp.var(x, axis=0, keepdims=True)
    x = jax.nn.relu((x - m) / jnp.sqrt(v + 1e-5) * g2 + b2)
    return x @ w3 + b3


# speedup vs baseline: 16.5649x; 15.2853x over previous
import functools

import jax
import jax.numpy as jnp
from jax import lax
from jax.experimental import pallas as pl
from jax.experimental.pallas import tpu as pltpu


def _fps_body(x_ref, out_ref, acc_ref, *, b, n, s):
    x0 = x_ref[0]
    x1 = x_ref[1]
    x2 = x_ref[2]
    iota_n = lax.broadcasted_iota(jnp.int32, (1, n), 1)
    iota_s = lax.broadcasted_iota(jnp.int32, (1, s), 1)
    acc_ref[...] = jnp.zeros((3, b, s), jnp.float32)

    def step(t, carry):
        dist, far = carry
        onehot = iota_n == far
        c0 = jnp.sum(jnp.where(onehot, x0, 0.0), axis=1, keepdims=True)
        c1 = jnp.sum(jnp.where(onehot, x1, 0.0), axis=1, keepdims=True)
        c2 = jnp.sum(jnp.where(onehot, x2, 0.0), axis=1, keepdims=True)
        sel = (iota_s == t).astype(jnp.float32)
        acc_ref[0] += c0 * sel
        acc_ref[1] += c1 * sel
        acc_ref[2] += c2 * sel
        d = (x0 - c0) ** 2 + (x1 - c1) ** 2 + (x2 - c2) ** 2
        dist = jnp.minimum(dist, d)
        far = jnp.argmax(dist, axis=1).astype(jnp.int32).reshape(b, 1)
        return dist, far

    init = (jnp.full((b, n), 1e10, jnp.float32), jnp.zeros((b, 1), jnp.int32))
    lax.fori_loop(0, s, step, init)
    out_ref[...] = acc_ref[...]


def _fps_pallas(xyz, s):
    b, n, _ = xyz.shape
    xyzt = jnp.transpose(xyz, (2, 0, 1))
    out = pl.pallas_call(
        functools.partial(_fps_body, b=b, n=n, s=s),
        in_specs=[pl.BlockSpec((3, b, n), lambda: (0, 0, 0))],
        out_specs=pl.BlockSpec((3, b, s), lambda: (0, 0, 0)),
        out_shape=jax.ShapeDtypeStruct((3, b, s), jnp.float32),
        scratch_shapes=[pltpu.VMEM((3, b, s), jnp.float32)],
    )(xyzt)
    return jnp.transpose(out, (1, 2, 0))


def _group_body(nx_ref, xt_ref, p_ref, out_ref, *, n, bs, nsample, r2, c_in):
    c = nx_ref[0]
    x = xt_ref[0]
    d = ((c[:, 0:1] - x[0:1, :]) ** 2
         + (c[:, 1:2] - x[1:2, :]) ** 2
         + (c[:, 2:3] - x[2:3, :]) ** 2)
    mask = d <= r2
    mi = mask.astype(jnp.int32)

    rank = mi
    sh = 1
    while sh < n:
        shifted = jnp.concatenate(
            [jnp.zeros((bs, sh), jnp.int32), rank[:, :n - sh]], axis=1)
        rank = rank + shifted
        sh *= 2

    cnt = rank[:, n - 1:n]
    iota_s = lax.broadcasted_iota(jnp.int32, (1, nsample), 1)
    tr = jnp.where(iota_s < cnt, iota_s + 1, 1)

    sel = (rank[:, None, :] == tr[:, :, None]) & mask[:, None, :]
    m = sel.astype(jnp.float32).reshape(bs * nsample, n)
    g = jnp.dot(m, p_ref[0], preferred_element_type=jnp.float32)
    g = g.reshape(bs, nsample, c_in)
    g = g - jnp.concatenate(
        [c, jnp.zeros((bs, c_in - 3), jnp.float32)], axis=1)[:, None, :]
    out_ref[0] = g


def _group_pallas(radius, nsample, xyz, new_xyz, pfeat):
    b, n, c_in = pfeat.shape
    s = new_xyz.shape[1]
    bs = 8
    import numpy as _np
    r2 = float(_np.float32(radius) * _np.float32(radius))
    xyzt = jnp.transpose(xyz, (0, 2, 1))
    return pl.pallas_call(
        functools.partial(_group_body, n=n, bs=bs, nsample=nsample, r2=r2,
                          c_in=c_in),
        grid=(b, s // bs),
        in_specs=[
            pl.BlockSpec((1, bs, 3), lambda i, j: (i, j, 0)),
            pl.BlockSpec((1, 3, n), lambda i, j: (i, 0, 0)),
            pl.BlockSpec((1, n, c_in), lambda i, j: (i, 0, 0)),
        ],
        out_specs=pl.BlockSpec((1, bs, nsample, c_in), lambda i, j: (i, j, 0, 0)),
        out_shape=jax.ShapeDtypeStruct((b, s, nsample, c_in), jnp.float32),
    )(new_xyz, xyzt, pfeat)


def _index_points(points, idx):
    def g(p, i):
        return p[i]

    return jax.vmap(g)(points, idx)


def _ball_query(radius, nsample, xyz, new_xyz):
    b, n, _ = xyz.shape
    s = new_xyz.shape[1]
    sqr = jnp.sum((new_xyz[:, :, None, :] - xyz[:, None, :, :]) ** 2, axis=-1)
    idx = jnp.broadcast_to(jnp.arange(n, dtype=jnp.int32), (b, s, n))
    idx = jnp.where(sqr > radius * radius, n, idx)
    idx = jnp.sort(idx, axis=-1)[:, :, :nsample]
    first = idx[:, :, :1]
    idx = jnp.where(idx == n, jnp.broadcast_to(first, idx.shape), idx)
    return idx


def _mlp(x, layers):
    for (w, g, bta) in layers:
        x = jnp.einsum('...c,cd->...d', x, w)
        m = jnp.mean(x, axis=tuple(range(x.ndim - 1)), keepdims=True)
        v = jnp.var(x, axis=tuple(range(x.ndim - 1)), keepdims=True)
        x = (x - m) / jnp.sqrt(v + 1e-5) * g + bta
        x = jax.nn.relu(x)
    return x


def _sa(xyz, feats, npoint, radius, nsample, layers):
    if npoint is None:
        grouped = xyz[:, None, :, :]
        if feats is not None:
            grouped = jnp.concatenate([grouped, feats[:, None, :, :]], axis=-1)
        new_xyz = jnp.zeros((xyz.shape[0], 1, 3), dtype=xyz.dtype)
    else:
        new_xyz = _fps_pallas(xyz, npoint)
        pfeat = jnp.concatenate([xyz, feats], axis=-1)
        grouped = _group_pallas(radius, nsample, xyz, new_xyz, pfeat)
    out = _mlp(grouped, layers)
    return new_xyz, jnp.max(out, axis=2)


def kernel(pointcloud, params):
    xyz = pointcloud[..., :3]
    feats = pointcloud[..., 3:]
    xyz, feats = _sa(xyz, feats, 512, 0.2, 64, params['sa1'])
    xyz, feats = _sa(xyz, feats, 128, 0.4, 64, params['sa2'])
    xyz, feats = _sa(xyz, feats, None, None, None, params['sa3'])
    x = feats[:, 0, :]
    w1, g1, b1, w2, g2, b2, w3, b3 = params['fc']
    x = x @ w1
    m = jnp.mean(x, axis=0, keepdims=True)
    v = jnp.var(x, axis=0, keepdims=True)
    x = jax.nn.relu((x - m) / jnp.sqrt(v + 1e-5) * g1 + b1)
    x = x @ w2
    m = jnp.mean(x, axis=0, keepdims=True)
    v = jnp.var(x, axis=0, keepdims=True)
    x = jax.nn.relu((x - m) / jnp.sqrt(v + 1e-5) * g2 + b2)
    return x @ w3 + b3
